# Initial kernel scaffold; baseline (speedup 1.0000x reference)
import jax
import jax.numpy as jnp
from jax.experimental import pallas as pl


def kernel(points_with_f_center, pillar_merge_coords, sparse_feat, points_indices_inv, W, gamma, beta):
    raise NotImplementedError("write your pallas kernel here")


# speedup vs baseline: 2.9300x; 2.9300x over previous
import functools
import jax
import jax.numpy as jnp
from jax import lax
from jax.experimental import pallas as pl
from jax.experimental.pallas import tpu as pltpu

P = 40000
PPAD = 40960
GX = 1440
GY = 1440
GZ = 8
COUT = 64
NPT = 4_194_304
SLOTS = 524_288
NREAL_SLOTS = 320_008
VCAP = 40_960


def _phase_T0_jnp(points_pad, W, gamma, beta):
    x = points_pad[:, 1:9]
    h = x @ W.T
    rowmask = (jnp.arange(PPAD) < P)
    hm = jnp.where(rowmask[:, None], h, 0.0)
    s1 = hm.sum(axis=0)
    s2 = (hm * hm).sum(axis=0)
    mu = s1 / P
    var = s2 / P - mu * mu
    g = gamma / jnp.sqrt(var + 1e-3)
    b2 = beta - g * mu
    zi = jnp.floor((points_pad[:, 6] + 2.0) / 0.5).astype(jnp.int32)
    zi = jnp.where(rowmask, zi, 0)
    coef = jnp.zeros((8, COUT), jnp.float32).at[0].set(g).at[1].set(b2)
    return h, coef, zi


def _phase_S1_jnp(pid_pad):
    occ = jnp.zeros((NPT,), jnp.int32).at[pid_pad].add(1)
    return occ


def _phase_T2_jnp(occ):
    return jnp.cumsum((occ != 0).astype(jnp.float32))


def _phase_S3_jnp(ypref, pid_pad, zi):
    yp = ypref[pid_pad]
    s = (yp.astype(jnp.int32) * 8 + zi)
    gpos = jnp.arange(PPAD)
    s = jnp.where(gpos < P, s, NREAL_SLOTS + 8 + (gpos - P))
    cslots = jnp.zeros((SLOTS,), jnp.float32).at[s].add(1.0)
    return cslots, s


def _phase_T4_jnp(cslots):
    occb = (cslots != 0).astype(jnp.float32)
    vrankp = jnp.cumsum(occb)
    occ8 = occb.reshape(SLOTS // 8, 8).sum(axis=1)
    return vrankp, occ8


def _phase_S5a_jnp(s, cslots, vrankp, sparse_feat):
    prank = (s >> 3) - 1
    prank_c = jnp.clip(prank, 0, P - 1)
    sfg = sparse_feat[prank_c]
    den = cslots[s]
    rcp = 1.0 / jnp.maximum(den, 1.0)
    den64 = jnp.broadcast_to(rcp[:, None], (PPAD, COUT))
    v = vrankp[s].astype(jnp.int32) - 1
    return sfg, den64, v


def _phase_T0B_jnp(h, coef, sfg, den64):
    g = coef[0]
    b2 = coef[1]
    bnr = jnp.maximum(g[None, :] * h + b2[None, :], 0.0)
    fs = (bnr + sfg) * den64
    rowmask = (jnp.arange(PPAD) < P)
    return jnp.where(rowmask[:, None], fs, 0.0)


def _phase_S5c_jnp(fs, v):
    vc = jnp.clip(v, 0, VCAP - 1)
    return jnp.zeros((VCAP, COUT), jnp.float32).at[vc].add(fs)


def _phase_S6_jnp(vox, s, v, shift):
    gpos = jnp.arange(PPAD)
    lidx = jnp.minimum(gpos, P - 1)
    s_c = s[lidx]
    v_c = v[lidx]
    f = s_c - 8 * (1 - shift)
    rows = vox[jnp.clip(v_c, 0, VCAP - 1)]
    out = jnp.zeros((P * 8, COUT), jnp.float32).at[f].set(rows)
    return out


def kernel(points_with_f_center, pillar_merge_coords, sparse_feat,
           points_indices_inv, W, gamma, beta):
    pts_pad = jnp.concatenate(
        [points_with_f_center,
         jnp.zeros((PPAD - P, 9), jnp.float32)], axis=0)
    gpos = jnp.arange(PPAD - P, dtype=jnp.int32)
    pid_fill = (2 * GX * GY) + (gpos % 1024)
    pid_pad = jnp.concatenate(
        [pillar_merge_coords.astype(jnp.int32), pid_fill], axis=0)

    h, coef, zi = _phase_T0_jnp(pts_pad, W, gamma, beta)
    occ = _phase_S1_jnp(pid_pad)
    ypref = _phase_T2_jnp(occ)
    cslots, s = _phase_S3_jnp(ypref, pid_pad, zi)
    vrankp, occ8 = _phase_T4_jnp(cslots)

    nv = vrankp[NREAL_SLOTS - 1].astype(jnp.int32)
    shift = jnp.where(nv < P, 1, 0).astype(jnp.int32)

    sfg, den64, v = _phase_S5a_jnp(s, cslots, vrankp, sparse_feat)
    fs = _phase_T0B_jnp(h, coef, sfg, den64)
    vox = _phase_S5c_jnp(fs, v)
    out = _phase_S6_jnp(vox, s, v, shift)

    src = out.reshape(P, 8, COUT)
    m1 = occ8[:P] >= 2
    m1 = m1.at[0].set((P - nv) >= 2)
    m0 = occ8[1:P + 1] >= 2
    occupied_mask = jnp.where(shift == 1, m1, m0)
    return src, occupied_mask


# speedup vs baseline: 4.0357x; 1.3774x over previous
import functools
import jax
import jax.numpy as jnp
from jax import lax
from jax.experimental import pallas as pl
from jax.experimental.pallas import tpu as pltpu

P = 40000
PPAD = 40960
GX = 1440
GY = 1440
GZ = 8
COUT = 64
NPT = 4_194_304
SLOTS = 524_288
NREAL_SLOTS = 320_008
VCAP = 40_960


def _prefix_body(x_ref, tri_ref, exc_ref, g_ref, y_ref, o8_ref, carry_ref):
    i = pl.program_id(0)

    @pl.when(i == 0)
    def _():
        carry_ref[0] = 0.0

    c0 = carry_ref[0]
    xb = (x_ref[...] != 0).astype(jnp.bfloat16)
    dn = (((1,), (0,)), ((), ()))
    y1 = lax.dot_general(xb, tri_ref[...], dn,
                         preferred_element_type=jnp.float32)
    rs = y1[:, 255:256]
    e = lax.dot_general(exc_ref[...], rs.astype(jnp.bfloat16), dn,
                        preferred_element_type=jnp.float32)
    y = y1 + e + c0
    y_ref[...] = y
    o8_ref[...] = lax.dot_general(xb, g_ref[...], dn,
                                  preferred_element_type=jnp.float32)
    carry_ref[0] = jnp.sum(lax.slice(y, (1023, 255), (1024, 256)))


def _prefix_pallas(x2d, interpret=False):
    nblk = x2d.shape[0] // 1024
    ar1k = jnp.arange(1024)
    tri = (jnp.arange(256)[:, None] <= jnp.arange(256)[None, :]
           ).astype(jnp.bfloat16)
    exc = (ar1k[:, None] > ar1k[None, :]).astype(jnp.bfloat16)
    gm = (jnp.arange(256)[:, None] // 8 == jnp.arange(32)[None, :]
          ).astype(jnp.bfloat16)
    return pl.pallas_call(
        _prefix_body,
        grid=(nblk,),
        in_specs=[
            pl.BlockSpec((1024, 256), lambda i: (i, 0)),
            pl.BlockSpec((256, 256), lambda i: (0, 0)),
            pl.BlockSpec((1024, 1024), lambda i: (0, 0)),
            pl.BlockSpec((256, 32), lambda i: (0, 0)),
        ],
        out_specs=[
            pl.BlockSpec((1024, 256), lambda i: (i, 0)),
            pl.BlockSpec((1024, 32), lambda i: (i, 0)),
        ],
        out_shape=[
            jax.ShapeDtypeStruct((nblk * 1024, 256), jnp.float32),
            jax.ShapeDtypeStruct((nblk * 1024, 32), jnp.float32),
        ],
        scratch_shapes=[pltpu.SMEM((1,), jnp.float32)],
        interpret=interpret,
    )(x2d, tri, exc, gm)


_NB0 = PPAD // 2560


def _t0_body(pts_ref, w_ref, gam_ref, bet_ref, h_ref, zi_ref, coef_ref,
             sums_ref):
    i = pl.program_id(0)

    @pl.when(i == 0)
    def _():
        sums_ref[...] = jnp.zeros_like(sums_ref)

    x = pts_ref[:, 1:9]
    dn = (((1,), (1,)), ((), ()))
    h = lax.dot_general(x, w_ref[...], dn, preferred_element_type=jnp.float32)
    h_ref[...] = h
    rows = i * 2560 + lax.broadcasted_iota(jnp.int32, (2560, 1), 0)
    hm = jnp.where(rows < P, h, 0.0)
    upd = jnp.concatenate([hm.sum(axis=0, keepdims=True),
                           (hm * hm).sum(axis=0, keepdims=True),
                           jnp.zeros((6, COUT), jnp.float32)], axis=0)
    sums_ref[...] += upd
    z = pts_ref[:, 6]
    zi = jnp.floor((z + 2.0) / 0.5).astype(jnp.int32)
    zi_ref[...] = zi.reshape(1, 1, 2560)

    @pl.when(i == _NB0 - 1)
    def _():
        s1 = sums_ref[0, :]
        s2 = sums_ref[1, :]
        mu = s1 / P
        var = s2 / P - mu * mu
        g = gam_ref[0, :] / jnp.sqrt(var + 1e-3)
        b2 = bet_ref[0, :] - g * mu
        coef_ref[...] = jnp.concatenate(
            [g.reshape(1, COUT), b2.reshape(1, COUT),
             jnp.zeros((6, COUT), jnp.float32)], axis=0)


def _phase_T0_pallas(points_pad, W, gamma, beta, interpret=False):
    return pl.pallas_call(
        _t0_body,
        grid=(_NB0,),
        in_specs=[
            pl.BlockSpec((2560, 9), lambda i: (i, 0)),
            pl.BlockSpec((COUT, 8), lambda i: (0, 0)),
            pl.BlockSpec((1, COUT), lambda i: (0, 0)),
            pl.BlockSpec((1, COUT), lambda i: (0, 0)),
        ],
        out_specs=[
            pl.BlockSpec((2560, COUT), lambda i: (i, 0)),
            pl.BlockSpec((1, 1, 2560), lambda i: (i, 0, 0)),
            pl.BlockSpec((8, COUT), lambda i: (0, 0)),
        ],
        out_shape=[
            jax.ShapeDtypeStruct((PPAD, COUT), jnp.float32),
            jax.ShapeDtypeStruct((_NB0, 1, 2560), jnp.int32),
            jax.ShapeDtypeStruct((8, COUT), jnp.float32),
        ],
        scratch_shapes=[pltpu.VMEM((8, COUT), jnp.float32)],
        interpret=interpret,
    )(points_pad, W, gamma.reshape(1, COUT), beta.reshape(1, COUT))


def _t0b_body(h_ref, coef_ref, sfg_ref, d_ref, fs_ref):
    i = pl.program_id(0)
    g = coef_ref[0:1, :]
    b2 = coef_ref[1:2, :]
    bnr = jnp.maximum(h_ref[...] * g + b2, 0.0)
    fs = (bnr + sfg_ref[...]) * d_ref[...]
    rows = i * 2560 + lax.broadcasted_iota(jnp.int32, (2560, 1), 0)
    fs_ref[...] = jnp.where(rows < P, fs, 0.0)


def _phase_T0B_pallas(h, coef, sfg, den64, interpret=False):
    bs = pl.BlockSpec((2560, COUT), lambda i: (i, 0))
    return pl.pallas_call(
        _t0b_body,
        grid=(_NB0,),
        in_specs=[bs, pl.BlockSpec((8, COUT), lambda i: (0, 0)), bs, bs],
        out_specs=bs,
        out_shape=jax.ShapeDtypeStruct((PPAD, COUT), jnp.float32),
        interpret=interpret,
    )(h, coef, sfg, den64)


def _tzero_body(o_ref):
    o_ref[...] = jnp.zeros_like(o_ref)


def _phase_Tzero_pallas(interpret=False):
    return pl.pallas_call(
        _tzero_body,
        grid=(40,),
        out_specs=pl.BlockSpec((8000, COUT), lambda i: (i, 0)),
        out_shape=jax.ShapeDtypeStruct((P * 8, COUT), jnp.float32),
        interpret=interpret,
    )()


def _phase_T0_jnp(points_pad, W, gamma, beta):
    x = points_pad[:, 1:9]
    h = x @ W.T
    rowmask = (jnp.arange(PPAD) < P)
    hm = jnp.where(rowmask[:, None], h, 0.0)
    s1 = hm.sum(axis=0)
    s2 = (hm * hm).sum(axis=0)
    mu = s1 / P
    var = s2 / P - mu * mu
    g = gamma / jnp.sqrt(var + 1e-3)
    b2 = beta - g * mu
    zi = jnp.floor((points_pad[:, 6] + 2.0) / 0.5).astype(jnp.int32)
    zi = jnp.where(rowmask, zi, 0)
    coef = jnp.zeros((8, COUT), jnp.float32).at[0].set(g).at[1].set(b2)
    return h, coef, zi


def _phase_S1_jnp(pid_pad):
    occ = jnp.zeros((NPT,), jnp.int32).at[pid_pad].add(1)
    return occ


def _phase_T2_jnp(occ):
    return jnp.cumsum((occ != 0).astype(jnp.float32))


def _phase_S3_jnp(ypref, pid_pad, zi):
    yp = ypref[pid_pad]
    s = (yp.astype(jnp.int32) * 8 + zi)
    gpos = jnp.arange(PPAD)
    s = jnp.where(gpos < P, s, NREAL_SLOTS + 8 + (gpos - P))
    cslots = jnp.zeros((SLOTS,), jnp.float32).at[s].add(1.0)
    return cslots, s


def _phase_T4_jnp(cslots):
    occb = (cslots != 0).astype(jnp.float32)
    vrankp = jnp.cumsum(occb)
    occ8 = occb.reshape(SLOTS // 8, 8).sum(axis=1)
    return vrankp, occ8


def _phase_S5a_jnp(s, cslots, vrankp, sparse_feat):
    prank = (s >> 3) - 1
    prank_c = jnp.clip(prank, 0, P - 1)
    sfg = sparse_feat[prank_c]
    den = cslots[s]
    rcp = 1.0 / jnp.maximum(den, 1.0)
    den64 = jnp.broadcast_to(rcp[:, None], (PPAD, COUT))
    v = vrankp[s].astype(jnp.int32) - 1
    return sfg, den64, v


def _phase_T0B_jnp(h, coef, sfg, den64):
    g = coef[0]
    b2 = coef[1]
    bnr = jnp.maximum(g[None, :] * h + b2[None, :], 0.0)
    fs = (bnr + sfg) * den64
    rowmask = (jnp.arange(PPAD) < P)
    return jnp.where(rowmask[:, None], fs, 0.0)


def _phase_S5c_jnp(fs, v):
    vc = jnp.clip(v, 0, VCAP - 1)
    return jnp.zeros((VCAP, COUT), jnp.float32).at[vc].add(fs)


def _phase_S6_jnp(vox, s, v, shift, out0):
    gpos = jnp.arange(PPAD)
    lidx = jnp.minimum(gpos, P - 1)
    s_c = s[lidx]
    v_c = v[lidx]
    f = s_c - 8 * (1 - shift)
    rows = vox[jnp.clip(v_c, 0, VCAP - 1)]
    out = out0.at[f].set(rows)
    return out


def kernel(points_with_f_center, pillar_merge_coords, sparse_feat,
           points_indices_inv, W, gamma, beta):
    pts_pad = jnp.concatenate(
        [points_with_f_center,
         jnp.zeros((PPAD - P, 9), jnp.float32)], axis=0)
    gpos = jnp.arange(PPAD - P, dtype=jnp.int32)
    pid_fill = (2 * GX * GY) + (gpos % 1024)
    pid_pad = jnp.concatenate(
        [pillar_merge_coords.astype(jnp.int32), pid_fill], axis=0)

    h, zi3d, coef = _phase_T0_pallas(pts_pad, W, gamma, beta)
    zi = zi3d.reshape(PPAD)
    occ = _phase_S1_jnp(pid_pad)
    y2d, _ = _prefix_pallas(occ.reshape(NPT // 256, 256))
    ypref = y2d.reshape(NPT)
    cslots, s = _phase_S3_jnp(ypref, pid_pad, zi)
    v2d, o82d = _prefix_pallas(cslots.reshape(SLOTS // 256, 256))
    vrankp = v2d.reshape(SLOTS)
    occ8 = o82d.reshape(SLOTS // 8)

    nv = vrankp[NREAL_SLOTS - 1].astype(jnp.int32)
    shift = jnp.where(nv < P, 1, 0).astype(jnp.int32)

    sfg, den64, v = _phase_S5a_jnp(s, cslots, vrankp, sparse_feat)
    fs = _phase_T0B_pallas(h, coef, sfg, den64)
    vox = _phase_S5c_jnp(fs, v)
    out0 = _phase_Tzero_pallas()
    out = _phase_S6_jnp(vox, s, v, shift, out0)

    src = out.reshape(P, 8, COUT)
    m1 = occ8[:P] >= 2
    m1 = m1.at[0].set((P - nv) >= 2)
    m0 = occ8[1:P + 1] >= 2
    occupied_mask = jnp.where(shift == 1, m1, m0)
    return src, occupied_mask


# speedup vs baseline: 5.2631x; 1.3041x over previous
import functools
import jax
import jax.numpy as jnp
from jax import lax
from jax.experimental import pallas as pl
from jax.experimental.pallas import tpu as pltpu
from jax.experimental.pallas import tpu_sc as plsc

P = 40000
PPAD = 40960
GX = 1440
GY = 1440
GZ = 8
COUT = 64
NPT = 4_147_200
SLOTS = 524_288
NREAL_SLOTS = 320_008
VCAP = 40_960
VCAP2 = 41_472


def _make_prefix_body(r):
    def _prefix_body(x_ref, tri_ref, exc_ref, g_ref, y_ref, o8_ref,
                     carry_ref):
        i = pl.program_id(0)

        @pl.when(i == 0)
        def _():
            carry_ref[0] = 0.0

        c0 = carry_ref[0]
        xb = (x_ref[...] != 0).astype(jnp.bfloat16)
        dn = (((1,), (0,)), ((), ()))
        y1 = lax.dot_general(xb, tri_ref[...], dn,
                             preferred_element_type=jnp.float32)
        rs = y1[:, 255:256]
        e = lax.dot_general(exc_ref[...], rs.astype(jnp.bfloat16), dn,
                            preferred_element_type=jnp.float32)
        y = y1 + e + c0
        y_ref[...] = y
        o8_ref[...] = lax.dot_general(xb, g_ref[...], dn,
                                      preferred_element_type=jnp.float32)
        carry_ref[0] = jnp.sum(lax.slice(y, (r - 1, 255), (r, 256)))
    return _prefix_body


def _prefix_pallas(x2d, r, interpret=False):
    nblk = x2d.shape[0] // r
    arr = jnp.arange(r)
    tri = (jnp.arange(256)[:, None] <= jnp.arange(256)[None, :]
           ).astype(jnp.bfloat16)
    exc = (arr[:, None] > arr[None, :]).astype(jnp.bfloat16)
    gm = (jnp.arange(256)[:, None] // 8 == jnp.arange(32)[None, :]
          ).astype(jnp.bfloat16)
    return pl.pallas_call(
        _make_prefix_body(r),
        grid=(nblk,),
        in_specs=[
            pl.BlockSpec((r, 256), lambda i: (i, 0)),
            pl.BlockSpec((256, 256), lambda i: (0, 0)),
            pl.BlockSpec((r, r), lambda i: (0, 0)),
            pl.BlockSpec((256, 32), lambda i: (0, 0)),
        ],
        out_specs=[
            pl.BlockSpec((r, 256), lambda i: (i, 0)),
            pl.BlockSpec((r, 32), lambda i: (i, 0)),
        ],
        out_shape=[
            jax.ShapeDtypeStruct((nblk * r, 256), jnp.float32),
            jax.ShapeDtypeStruct((nblk * r, 32), jnp.float32),
        ],
        scratch_shapes=[pltpu.SMEM((1,), jnp.float32)],
        interpret=interpret,
    )(x2d, tri, exc, gm)


_NB0 = PPAD // 2560


def _t0_body(pts_ref, w_ref, gam_ref, bet_ref, h_ref, zi_ref, coef_ref,
             sums_ref):
    i = pl.program_id(0)

    @pl.when(i == 0)
    def _():
        sums_ref[...] = jnp.zeros_like(sums_ref)

    x = pts_ref[:, 1:9]
    dn = (((1,), (1,)), ((), ()))
    h = lax.dot_general(x, w_ref[...], dn, preferred_element_type=jnp.float32)
    h_ref[...] = h
    rows = i * 2560 + lax.broadcasted_iota(jnp.int32, (2560, 1), 0)
    hm = jnp.where(rows < P, h, 0.0)
    upd = jnp.concatenate([hm.sum(axis=0, keepdims=True),
                           (hm * hm).sum(axis=0, keepdims=True),
                           jnp.zeros((6, COUT), jnp.float32)], axis=0)
    sums_ref[...] += upd
    z = pts_ref[:, 6]
    zi = jnp.floor((z + 2.0) / 0.5).astype(jnp.int32)
    zi_ref[...] = zi.reshape(1, 1, 2560)

    @pl.when(i == _NB0 - 1)
    def _():
        s1 = sums_ref[0, :]
        s2 = sums_ref[1, :]
        mu = s1 / P
        var = s2 / P - mu * mu
        g = gam_ref[0, :] / jnp.sqrt(var + 1e-3)
        b2 = bet_ref[0, :] - g * mu
        coef_ref[...] = jnp.concatenate(
            [g.reshape(1, COUT), b2.reshape(1, COUT),
             jnp.zeros((6, COUT), jnp.float32)], axis=0)


def _phase_T0_pallas(points_pad, W, gamma, beta, interpret=False):
    return pl.pallas_call(
        _t0_body,
        grid=(_NB0,),
        in_specs=[
            pl.BlockSpec((2560, 9), lambda i: (i, 0)),
            pl.BlockSpec((COUT, 8), lambda i: (0, 0)),
            pl.BlockSpec((1, COUT), lambda i: (0, 0)),
            pl.BlockSpec((1, COUT), lambda i: (0, 0)),
        ],
        out_specs=[
            pl.BlockSpec((2560, COUT), lambda i: (i, 0)),
            pl.BlockSpec((1, 1, 2560), lambda i: (i, 0, 0)),
            pl.BlockSpec((8, COUT), lambda i: (0, 0)),
        ],
        out_shape=[
            jax.ShapeDtypeStruct((PPAD, COUT), jnp.float32),
            jax.ShapeDtypeStruct((_NB0, 1, 2560), jnp.int32),
            jax.ShapeDtypeStruct((8, COUT), jnp.float32),
        ],
        scratch_shapes=[pltpu.VMEM((8, COUT), jnp.float32)],
        interpret=interpret,
    )(points_pad, W, gamma.reshape(1, COUT), beta.reshape(1, COUT))


def _t0b_body(h_ref, coef_ref, sfg_ref, d_ref, pp_ref, ps_ref, fs_ref):
    i = pl.program_id(0)
    g = coef_ref[0:1, :]
    b2 = coef_ref[1:2, :]
    bnr = jnp.maximum(h_ref[...] * g + b2, 0.0)
    sf2 = sfg_ref[...]
    sf = jnp.where(pp_ref[...] == 0, sf2[:, :COUT], sf2[:, COUT:])
    fs = (bnr + sf) * d_ref[...]
    rows = i * 2560 + lax.broadcasted_iota(jnp.int32, (2560, 1), 0)
    fs = jnp.where(rows < P, fs, 0.0)
    even = ps_ref[...] == 0
    fs_ref[...] = jnp.concatenate(
        [jnp.where(even, fs, 0.0), jnp.where(even, 0.0, fs)], axis=1)


def _phase_T0B_pallas(h, coef, sfg2, rcp2d, parp2d, pars2d, interpret=False):
    bs = pl.BlockSpec((2560, COUT), lambda i: (i, 0))
    bs128 = pl.BlockSpec((2560, 128), lambda i: (i, 0))
    bs1 = pl.BlockSpec((2560, 1), lambda i: (i, 0))
    return pl.pallas_call(
        _t0b_body,
        grid=(_NB0,),
        in_specs=[bs, pl.BlockSpec((8, COUT), lambda i: (0, 0)), bs128,
                  bs1, bs1, bs1],
        out_specs=bs128,
        out_shape=jax.ShapeDtypeStruct((PPAD, 128), jnp.float32),
        interpret=interpret,
    )(h, coef, sfg2, rcp2d, parp2d, pars2d)


def _tzero_body(o_ref):
    o_ref[...] = jnp.zeros_like(o_ref)


def _phase_Tzero_pallas(interpret=False):
    return pl.pallas_call(
        _tzero_body,
        grid=(40,),
        out_specs=pl.BlockSpec((8000, COUT), lambda i: (i, 0)),
        out_shape=jax.ShapeDtypeStruct((P * 8, COUT), jnp.float32),
        interpret=interpret,
    )()


_QUARTER = NPT // 4
_TRASH = 16384
_SHALF = SLOTS // 2
_L = 16


def _sc_mesh():
    return plsc.VectorSubcoreMesh(core_axis_name="c", subcore_axis_name="s")


def _iota16():
    return lax.iota(jnp.int32, 16)


def _phase_S1_sc(pid_pad):
    zc = jnp.zeros((16200,), jnp.int32)
    on = jnp.ones((128,), jnp.int32)

    @functools.partial(
        pl.kernel, mesh=_sc_mesh(),
        out_type=jax.ShapeDtypeStruct((NPT,), jnp.int32),
        scratch_types=[
            pltpu.VMEM_SHARED((_QUARTER + _TRASH,), jnp.int32),
            pltpu.VMEM((2560,), jnp.int32),
            pltpu.VMEM((2560,), jnp.int32),
            pltpu.VMEM((16200,), jnp.int32),
            pltpu.VMEM((16200,), jnp.int32),
            pltpu.VMEM((128,), jnp.int32),
        ],
    )
    def k(pid_hbm, zc_hbm, on_hbm, occ_hbm, counts, pidv, idxv, zbuf, wbuf,
          ones):
        c = lax.axis_index("c")
        t = lax.axis_index("s")
        pltpu.sync_copy(zc_hbm, zbuf)
        pltpu.sync_copy(on_hbm, ones)
        pltpu.sync_copy(pid_hbm.at[pl.ds(t * 2560, 2560)], pidv)
        for half in range(2):
            base = half * 2 * _QUARTER + c * _QUARTER
            for j in range(4):
                pltpu.sync_copy(zbuf, counts.at[pl.ds(t * 64800 + j * 16200,
                                                      16200)])
            plsc.subcore_barrier()
            def mk(k_, _):
                vec = pidv[pl.ds(k_ * 16, 16)]
                local = vec - base
                inr = (local >= 0) & (local < _QUARTER)
                sp = _QUARTER + ((k_ * 16 + _iota16()) & (_TRASH - 1))
                idx = jnp.where(inr, local, sp)
                idxv[pl.ds(k_ * 16, 16)] = idx
                return 0
            lax.fori_loop(0, 160, mk, 0, unroll=8)
            for ck in range(20):
                pltpu.sync_copy(ones,
                                counts.at[idxv.at[pl.ds(ck * 128, 128)]],
                                add=True)
            plsc.subcore_barrier()
            for j in range(4):
                sl = pl.ds(t * 64800 + j * 16200, 16200)
                pltpu.sync_copy(counts.at[sl], wbuf)
                pltpu.sync_copy(
                    wbuf, occ_hbm.at[pl.ds(base + t * 64800 + j * 16200,
                                           16200)])
            if half == 0:
                plsc.subcore_barrier()

    return k(pid_pad, zc, on)


def _phase_S3_sc(ypref, pid_pad, zi):
    zc32 = jnp.zeros((16384,), jnp.float32)
    on32 = jnp.ones((128,), jnp.float32)

    @functools.partial(
        pl.kernel, mesh=_sc_mesh(),
        out_type=[jax.ShapeDtypeStruct((SLOTS,), jnp.float32),
                  jax.ShapeDtypeStruct((PPAD,), jnp.int32)],
        scratch_types=[
            pltpu.VMEM_SHARED((_SHALF + _TRASH,), jnp.float32),
            pltpu.VMEM((2560,), jnp.int32),
            pltpu.VMEM((2560,), jnp.int32),
            pltpu.VMEM((2560,), jnp.int32),
            pltpu.VMEM((2560,), jnp.int32),
            pltpu.VMEM((128,), jnp.float32),
            pltpu.VMEM((16384,), jnp.float32),
            pltpu.VMEM((128,), jnp.float32),
        ],
    )
    def k(yp_hbm, pid_hbm, zi_hbm, zc_hbm, on_hbm, cs_hbm, s_hbm, slots,
          pidv, ziv, sv, sidx, ypv, zbuf, ones):
        c = lax.axis_index("c")
        t = lax.axis_index("s")
        pltpu.sync_copy(zc_hbm, zbuf)
        pltpu.sync_copy(on_hbm, ones)
        pltpu.sync_copy(zbuf, slots.at[pl.ds(t * 16384, 16384)])
        plsc.subcore_barrier()
        pltpu.sync_copy(pid_hbm.at[pl.ds(t * 2560, 2560)], pidv)
        pltpu.sync_copy(zi_hbm.at[pl.ds(t * 2560, 2560)], ziv)
        base = t * 2560
        half_lo = c * _SHALF
        def clampg(l, _):
            vec = pidv[pl.ds(l * 16, 16)]
            pidv[pl.ds(l * 16, 16)] = jnp.minimum(vec, NPT - 1)
            return 0
        lax.fori_loop(0, 160, clampg, 0, unroll=8)
        for ck in range(20):
            pltpu.sync_copy(yp_hbm.at[pidv.at[pl.ds(ck * 128, 128)]], ypv)
            def mks(l, _):
                y = ypv[pl.ds(l * 16, 16)]
                z = ziv[pl.ds(ck * 128 + l * 16, 16)]
                s = y.astype(jnp.int32) * 8 + z
                gpos = base + ck * 128 + l * 16 + _iota16()
                s = jnp.where(gpos < P,
                              s, NREAL_SLOTS + 8 + (gpos & 8191))
                sv[pl.ds(ck * 128 + l * 16, 16)] = s
                local = s - half_lo
                inr = (local >= 0) & (local < _SHALF)
                sp = _SHALF + ((ck * 128 + l * 16 + _iota16())
                               & (_TRASH - 1))
                sidx[pl.ds(ck * 128 + l * 16, 16)] = jnp.where(inr, local, sp)
                return 0
            lax.fori_loop(0, 8, mks, 0, unroll=8)
            pltpu.sync_copy(ones, slots.at[sidx.at[pl.ds(ck * 128, 128)]],
                            add=True)

        @pl.when(c == 0)
        def _():
            pltpu.sync_copy(sv, s_hbm.at[pl.ds(t * 2560, 2560)])
        plsc.subcore_barrier()
        sl = pl.ds(t * 16384, 16384)
        pltpu.sync_copy(slots.at[sl], zbuf)
        pltpu.sync_copy(zbuf, cs_hbm.at[pl.ds(c * _SHALF + t * 16384,
                                              16384)])

    return k(ypref, pid_pad, zi, zc32, on32)


def _phase_S5a_sc(s, cslots, vp2, sf2):
    @functools.partial(
        pl.kernel, mesh=_sc_mesh(),
        out_type=[jax.ShapeDtypeStruct((PPAD, 128), jnp.float32),
                  jax.ShapeDtypeStruct((PPAD,), jnp.float32),
                  jax.ShapeDtypeStruct((PPAD,), jnp.int32),
                  jax.ShapeDtypeStruct((PPAD,), jnp.int32),
                  jax.ShapeDtypeStruct((PPAD,), jnp.int32)],
        scratch_types=[
            pltpu.VMEM((1280,), jnp.int32),
            pltpu.VMEM((1280,), jnp.int32),
            pltpu.VMEM((1280,), jnp.int32),
            pltpu.VMEM((1280,), jnp.int32),
            pltpu.VMEM((1280,), jnp.int32),
            pltpu.VMEM((1280,), jnp.int32),
            pltpu.VMEM((128, 128), jnp.float32),
            pltpu.VMEM((128,), jnp.float32),
            pltpu.VMEM((1280,), jnp.float32),
        ],
    )
    def k(s_hbm, cs_hbm, vp2_hbm, sf2_hbm, sfg_hbm, rcp_hbm, parp_hbm,
          pars_hbm, v2_hbm, sv, rowv, spv, vv, parpv, parsv, rows, val, rcp):
        c = lax.axis_index("c")
        t = lax.axis_index("s")
        w = t * 2 + c
        b0 = w * 1280
        pltpu.sync_copy(s_hbm.at[pl.ds(b0, 1280)], sv)

        def prep(l, _):
            vec = sv[pl.ds(l * 16, 16)]
            pr = jnp.clip((vec >> 3) - 1, 0, P - 1)
            rowv[pl.ds(l * 16, 16)] = pr >> 1
            parpv[pl.ds(l * 16, 16)] = pr & 1
            spv[pl.ds(l * 16, 16)] = vec >> 1
            parsv[pl.ds(l * 16, 16)] = vec & 1
            return 0
        lax.fori_loop(0, 80, prep, 0, unroll=8)
        for ck in range(10):
            pltpu.sync_copy(sf2_hbm.at[rowv.at[pl.ds(ck * 128, 128)]], rows)
            pltpu.sync_copy(rows, sfg_hbm.at[pl.ds(b0 + ck * 128, 128)])
            pltpu.sync_copy(cs_hbm.at[sv.at[pl.ds(ck * 128, 128)]], val)

            def mkrcp(l, _):
                d = val[pl.ds(l * 16, 16)]
                rcp[pl.ds(ck * 128 + l * 16, 16)] = 1.0 / jnp.maximum(d, 1.0)
                return 0
            lax.fori_loop(0, 8, mkrcp, 0, unroll=8)
            pltpu.sync_copy(vp2_hbm.at[spv.at[pl.ds(ck * 128, 128)]], val)

            def mkv(l, _):
                vv[pl.ds(ck * 128 + l * 16, 16)] = (
                    val[pl.ds(l * 16, 16)].astype(jnp.int32) - 1)
                return 0
            lax.fori_loop(0, 8, mkv, 0, unroll=8)
        pltpu.sync_copy(vv, v2_hbm.at[pl.ds(b0, 1280)])
        pltpu.sync_copy(rcp, rcp_hbm.at[pl.ds(b0, 1280)])
        pltpu.sync_copy(parpv, parp_hbm.at[pl.ds(b0, 1280)])
        pltpu.sync_copy(parsv, pars_hbm.at[pl.ds(b0, 1280)])

    sfg2, rcpf, parp, pars, v2 = k(s, cslots, vp2, sf2)
    return (sfg2, rcpf.reshape(PPAD, 1), parp.reshape(PPAD, 1),
            pars.reshape(PPAD, 1), v2)


def _phase_S5c_sc(fs2, v2):
    _VQ = VCAP2 // 4
    zc2 = jnp.zeros((72, 128), jnp.float32)

    @functools.partial(
        pl.kernel, mesh=_sc_mesh(),
        out_type=jax.ShapeDtypeStruct((VCAP2, 128), jnp.float32),
        scratch_types=[
            pltpu.VMEM_SHARED((_VQ + 64, 128), jnp.float32),
            pltpu.VMEM((2560,), jnp.int32),
            pltpu.VMEM((2560,), jnp.int32),
            pltpu.VMEM((72, 128), jnp.float32),
            pltpu.VMEM((72, 128), jnp.float32),
            pltpu.VMEM((128, 128), jnp.float32),
        ],
    )
    def k(fs_hbm, v_hbm, zc_hbm, vox_hbm, acc, vv, idxv, zb, wb, rows):
        c = lax.axis_index("c")
        t = lax.axis_index("s")
        pltpu.sync_copy(zc_hbm, zb)
        pltpu.sync_copy(v_hbm.at[pl.ds(t * 2560, 2560)], vv)
        for ps in range(2):
            qbase = (2 * ps + c) * _VQ
            for j in range(9):
                pltpu.sync_copy(zb, acc.at[pl.ds(t * 648 + j * 72, 72)])
            plsc.subcore_barrier()

            def mk(l, _):
                vec = vv[pl.ds(l * 16, 16)]
                local = vec - qbase
                inr = (local >= 0) & (local < _VQ)
                sp = _VQ + ((l * 16 + _iota16()) & 63)
                idxv[pl.ds(l * 16, 16)] = jnp.where(inr, local, sp)
                return 0
            lax.fori_loop(0, 160, mk, 0, unroll=8)
            for ck in range(20):
                pltpu.sync_copy(fs_hbm.at[pl.ds(t * 2560 + ck * 128, 128)],
                                rows)
                pltpu.sync_copy(rows, acc.at[idxv.at[pl.ds(ck * 128, 128)]],
                                add=True)
            plsc.subcore_barrier()
            for j in range(9):
                pltpu.sync_copy(acc.at[pl.ds(t * 648 + j * 72, 72)], wb)
                pltpu.sync_copy(wb, vox_hbm.at[pl.ds(qbase + t * 648
                                                     + j * 72, 72)])
            if ps == 0:
                plsc.subcore_barrier()

    return k(fs2, v2, zc2)


def _phase_S6_sc(cp_shifted, vp2_shifted, vox2):
    _NCH = (P * 8 // 2) // 128

    @functools.partial(
        pl.kernel, mesh=_sc_mesh(),
        out_type=jax.ShapeDtypeStruct((P * 8 // 2, 128), jnp.float32),
        scratch_types=[
            pltpu.VMEM((128,), jnp.float32),
            pltpu.VMEM((128,), jnp.float32),
            pltpu.VMEM((128,), jnp.int32),
            pltpu.VMEM((128, 128), jnp.float32),
        ],
    )
    def k(cp_hbm, vp_hbm, vox_hbm, out_hbm, cpv, vpv, idxv, rows):
        c = lax.axis_index("c")
        t = lax.axis_index("s")
        w = t * 2 + c

        def chunk(kk, _):
            ck = w + kk * 32

            @pl.when(ck < _NCH)
            def _():
                r0 = ck * 128
                pltpu.sync_copy(cp_hbm.at[pl.ds(r0, 128)], cpv)
                pltpu.sync_copy(vp_hbm.at[pl.ds(r0, 128)], vpv)

                def mk(l, _2):
                    occ = cpv[pl.ds(l * 16, 16)] > 0.0
                    vr = vpv[pl.ds(l * 16, 16)].astype(jnp.int32) - 1
                    vr = jnp.clip(vr, 0, VCAP2 - 1)
                    pos = r0 + l * 16 + _iota16()
                    sp = 40000 + ((pos * 7 + l) & 1023)
                    idxv[pl.ds(l * 16, 16)] = jnp.where(occ, vr, sp)
                    return 0
                lax.fori_loop(0, 8, mk, 0, unroll=8)
                pltpu.sync_copy(vox_hbm.at[idxv], rows)
                pltpu.sync_copy(rows, out_hbm.at[pl.ds(r0, 128)])
            return 0
        lax.fori_loop(0, 40, chunk, 0)

    return k(cp_shifted, vp2_shifted, vox2)


def _phase_T0_jnp(points_pad, W, gamma, beta):
    x = points_pad[:, 1:9]
    h = x @ W.T
    rowmask = (jnp.arange(PPAD) < P)
    hm = jnp.where(rowmask[:, None], h, 0.0)
    s1 = hm.sum(axis=0)
    s2 = (hm * hm).sum(axis=0)
    mu = s1 / P
    var = s2 / P - mu * mu
    g = gamma / jnp.sqrt(var + 1e-3)
    b2 = beta - g * mu
    zi = jnp.floor((points_pad[:, 6] + 2.0) / 0.5).astype(jnp.int32)
    zi = jnp.where(rowmask, zi, 0)
    coef = jnp.zeros((8, COUT), jnp.float32).at[0].set(g).at[1].set(b2)
    return h, coef, zi


def _phase_S1_jnp(pid_pad):
    occ = jnp.zeros((NPT,), jnp.int32).at[pid_pad].add(1)
    return occ


def _phase_T2_jnp(occ):
    return jnp.cumsum((occ != 0).astype(jnp.float32))


def _phase_S3_jnp(ypref, pid_pad, zi):
    yp = ypref[pid_pad]
    s = (yp.astype(jnp.int32) * 8 + zi)
    gpos = jnp.arange(PPAD)
    s = jnp.where(gpos < P, s, NREAL_SLOTS + 8 + (gpos - P))
    cslots = jnp.zeros((SLOTS,), jnp.float32).at[s].add(1.0)
    return cslots, s


def _phase_T4_jnp(cslots):
    occb = (cslots != 0).astype(jnp.float32)
    vrankp = jnp.cumsum(occb)
    occ8 = occb.reshape(SLOTS // 8, 8).sum(axis=1)
    return vrankp, occ8


def _phase_S5a_jnp(s, cslots, vrankp, sparse_feat):
    prank = (s >> 3) - 1
    prank_c = jnp.clip(prank, 0, P - 1)
    sfg = sparse_feat[prank_c]
    den = cslots[s]
    rcp = 1.0 / jnp.maximum(den, 1.0)
    den64 = jnp.broadcast_to(rcp[:, None], (PPAD, COUT))
    v = vrankp[s].astype(jnp.int32) - 1
    return sfg, den64, v


def _phase_T0B_jnp(h, coef, sfg, den64):
    g = coef[0]
    b2 = coef[1]
    bnr = jnp.maximum(g[None, :] * h + b2[None, :], 0.0)
    fs = (bnr + sfg) * den64
    rowmask = (jnp.arange(PPAD) < P)
    return jnp.where(rowmask[:, None], fs, 0.0)


def _phase_S5c_jnp(fs, v):
    vc = jnp.clip(v, 0, VCAP - 1)
    return jnp.zeros((VCAP, COUT), jnp.float32).at[vc].add(fs)


def _phase_S6_jnp(vox, s, v, shift, out0):
    gpos = jnp.arange(PPAD)
    lidx = jnp.minimum(gpos, P - 1)
    s_c = s[lidx]
    v_c = v[lidx]
    f = s_c - 8 * (1 - shift)
    rows = vox[jnp.clip(v_c, 0, VCAP - 1)]
    out = out0.at[f].set(rows)
    return out


def kernel(points_with_f_center, pillar_merge_coords, sparse_feat,
           points_indices_inv, W, gamma, beta):
    pts_pad = jnp.concatenate(
        [points_with_f_center,
         jnp.zeros((PPAD - P, 9), jnp.float32)], axis=0)
    pid_fill = jnp.full((PPAD - P,), 1 << 29, jnp.int32)
    pid_pad = jnp.concatenate(
        [pillar_merge_coords.astype(jnp.int32), pid_fill], axis=0)

    h, zi3d, coef = _phase_T0_pallas(pts_pad, W, gamma, beta)
    zi = zi3d.reshape(PPAD)
    occ = _phase_S1_sc(pid_pad)
    y2d, _ = _prefix_pallas(occ.reshape(NPT // 256, 256), 600)
    ypref = y2d.reshape(NPT)
    cslots, s = _phase_S3_sc(ypref, pid_pad, zi)
    v2d, o82d = _prefix_pallas(cslots.reshape(SLOTS // 256, 256), 1024)
    vrankp = v2d.reshape(SLOTS)
    occ8 = o82d.reshape(SLOTS // 8)

    nv = vrankp[NREAL_SLOTS - 1].astype(jnp.int32)
    shift = jnp.where(nv < P, 1, 0).astype(jnp.int32)

    cpair = cslots.reshape(SLOTS // 2, 2).sum(axis=1)
    vp2_2d, _ = _prefix_pallas(cpair.reshape(SLOTS // 512, 256), 1024)
    vp2 = vp2_2d.reshape(SLOTS // 2)
    sf2 = sparse_feat.reshape(P // 2, 128)

    sfg2, rcp2d, parp2d, pars2d, v2 = _phase_S5a_sc(s, cslots, vp2, sf2)
    fs2 = _phase_T0B_pallas(h, coef, sfg2, rcp2d, parp2d, pars2d)
    vox2 = _phase_S5c_sc(fs2, v2)
    d2 = 4 * (1 - shift)
    cp_sh = lax.dynamic_slice(cpair, (d2,), (P * 4,))
    vp2_sh = lax.dynamic_slice(vp2, (d2,), (P * 4,))
    out = _phase_S6_sc(cp_sh, vp2_sh, vox2)

    src = out.reshape(P, 8, COUT)
    m1 = occ8[:P] >= 2
    m1 = m1.at[0].set((P - nv) >= 2)
    m0 = occ8[1:P + 1] >= 2
    occupied_mask = jnp.where(shift == 1, m1, m0)
    return src, occupied_mask


# speedup vs baseline: 5.4391x; 1.0334x over previous
import functools
import jax
import jax.numpy as jnp
from jax import lax
from jax.experimental import pallas as pl
from jax.experimental.pallas import tpu as pltpu
from jax.experimental.pallas import tpu_sc as plsc

P = 40000
PPAD = 40960
GX = 1440
GY = 1440
GZ = 8
COUT = 64
NPT = 4_147_200
SLOTS = 524_288
NREAL_SLOTS = 320_008
VCAP = 40_960
VCAP2 = 41_472


def _make_prefix_body(r):
    def _prefix_body(x_ref, tri_ref, exc_ref, g_ref, y_ref, o8_ref,
                     carry_ref):
        i = pl.program_id(0)

        @pl.when(i == 0)
        def _():
            carry_ref[0] = 0.0

        c0 = carry_ref[0]
        xb = (x_ref[...] != 0).astype(jnp.bfloat16)
        dn = (((1,), (0,)), ((), ()))
        y1 = lax.dot_general(xb, tri_ref[...], dn,
                             preferred_element_type=jnp.float32)
        rs = y1[:, 255:256]
        e = lax.dot_general(exc_ref[...], rs.astype(jnp.bfloat16), dn,
                            preferred_element_type=jnp.float32)
        y = y1 + e + c0
        y_ref[...] = y
        o8_ref[...] = lax.dot_general(xb, g_ref[...], dn,
                                      preferred_element_type=jnp.float32)
        carry_ref[0] = jnp.sum(lax.slice(y, (r - 1, 255), (r, 256)))
    return _prefix_body


def _prefix_pallas(x2d, r, interpret=False):
    nblk = x2d.shape[0] // r
    arr = jnp.arange(r)
    tri = (jnp.arange(256)[:, None] <= jnp.arange(256)[None, :]
           ).astype(jnp.bfloat16)
    exc = (arr[:, None] > arr[None, :]).astype(jnp.bfloat16)
    gm = (jnp.arange(256)[:, None] // 8 == jnp.arange(32)[None, :]
          ).astype(jnp.bfloat16)
    return pl.pallas_call(
        _make_prefix_body(r),
        grid=(nblk,),
        in_specs=[
            pl.BlockSpec((r, 256), lambda i: (i, 0)),
            pl.BlockSpec((256, 256), lambda i: (0, 0)),
            pl.BlockSpec((r, r), lambda i: (0, 0)),
            pl.BlockSpec((256, 32), lambda i: (0, 0)),
        ],
        out_specs=[
            pl.BlockSpec((r, 256), lambda i: (i, 0)),
            pl.BlockSpec((r, 32), lambda i: (i, 0)),
        ],
        out_shape=[
            jax.ShapeDtypeStruct((nblk * r, 256), jnp.float32),
            jax.ShapeDtypeStruct((nblk * r, 32), jnp.float32),
        ],
        scratch_shapes=[pltpu.SMEM((1,), jnp.float32)],
        interpret=interpret,
    )(x2d, tri, exc, gm)


_NB0 = PPAD // 2560


def _t0_body(pts_ref, w_ref, gam_ref, bet_ref, h_ref, zi_ref, coef_ref,
             sums_ref):
    i = pl.program_id(0)

    @pl.when(i == 0)
    def _():
        sums_ref[...] = jnp.zeros_like(sums_ref)

    x = pts_ref[:, 1:9]
    dn = (((1,), (1,)), ((), ()))
    h = lax.dot_general(x, w_ref[...], dn, preferred_element_type=jnp.float32)
    h_ref[...] = h
    rows = i * 2560 + lax.broadcasted_iota(jnp.int32, (2560, 1), 0)
    hm = jnp.where(rows < P, h, 0.0)
    upd = jnp.concatenate([hm.sum(axis=0, keepdims=True),
                           (hm * hm).sum(axis=0, keepdims=True),
                           jnp.zeros((6, COUT), jnp.float32)], axis=0)
    sums_ref[...] += upd
    z = pts_ref[:, 6]
    zi = jnp.floor((z + 2.0) / 0.5).astype(jnp.int32)
    zi_ref[...] = zi.reshape(1, 1, 2560)

    @pl.when(i == _NB0 - 1)
    def _():
        s1 = sums_ref[0, :]
        s2 = sums_ref[1, :]
        mu = s1 / P
        var = s2 / P - mu * mu
        g = gam_ref[0, :] / jnp.sqrt(var + 1e-3)
        b2 = bet_ref[0, :] - g * mu
        coef_ref[...] = jnp.concatenate(
            [g.reshape(1, COUT), b2.reshape(1, COUT),
             jnp.zeros((6, COUT), jnp.float32)], axis=0)


def _phase_T0_pallas(points_pad, W, gamma, beta, interpret=False):
    return pl.pallas_call(
        _t0_body,
        grid=(_NB0,),
        in_specs=[
            pl.BlockSpec((2560, 9), lambda i: (i, 0)),
            pl.BlockSpec((COUT, 8), lambda i: (0, 0)),
            pl.BlockSpec((1, COUT), lambda i: (0, 0)),
            pl.BlockSpec((1, COUT), lambda i: (0, 0)),
        ],
        out_specs=[
            pl.BlockSpec((2560, COUT), lambda i: (i, 0)),
            pl.BlockSpec((1, 1, 2560), lambda i: (i, 0, 0)),
            pl.BlockSpec((8, COUT), lambda i: (0, 0)),
        ],
        out_shape=[
            jax.ShapeDtypeStruct((PPAD, COUT), jnp.float32),
            jax.ShapeDtypeStruct((_NB0, 1, 2560), jnp.int32),
            jax.ShapeDtypeStruct((8, COUT), jnp.float32),
        ],
        scratch_shapes=[pltpu.VMEM((8, COUT), jnp.float32)],
        interpret=interpret,
    )(points_pad, W, gamma.reshape(1, COUT), beta.reshape(1, COUT))


def _t0b_body(h_ref, coef_ref, sfg_ref, d_ref, pp_ref, ps_ref, fs_ref):
    i = pl.program_id(0)
    g = coef_ref[0:1, :]
    b2 = coef_ref[1:2, :]
    bnr = jnp.maximum(h_ref[...] * g + b2, 0.0)
    sf2 = sfg_ref[...]
    sf = jnp.where(pp_ref[...] == 0, sf2[:, :COUT], sf2[:, COUT:])
    fs = (bnr + sf) * d_ref[...]
    rows = i * 2560 + lax.broadcasted_iota(jnp.int32, (2560, 1), 0)
    fs = jnp.where(rows < P, fs, 0.0)
    even = ps_ref[...] == 0
    fs_ref[...] = jnp.concatenate(
        [jnp.where(even, fs, 0.0), jnp.where(even, 0.0, fs)], axis=1)


def _phase_T0B_pallas(h, coef, sfg2, rcp2d, parp2d, pars2d, interpret=False):
    bs = pl.BlockSpec((2560, COUT), lambda i: (i, 0))
    bs128 = pl.BlockSpec((2560, 128), lambda i: (i, 0))
    bs1 = pl.BlockSpec((2560, 1), lambda i: (i, 0))
    return pl.pallas_call(
        _t0b_body,
        grid=(_NB0,),
        in_specs=[bs, pl.BlockSpec((8, COUT), lambda i: (0, 0)), bs128,
                  bs1, bs1, bs1],
        out_specs=bs128,
        out_shape=jax.ShapeDtypeStruct((PPAD, 128), jnp.float32),
        interpret=interpret,
    )(h, coef, sfg2, rcp2d, parp2d, pars2d)


def _tzero_body(o_ref):
    o_ref[...] = jnp.zeros_like(o_ref)


def _phase_Tzero_pallas(interpret=False):
    return pl.pallas_call(
        _tzero_body,
        grid=(40,),
        out_specs=pl.BlockSpec((8000, COUT), lambda i: (i, 0)),
        out_shape=jax.ShapeDtypeStruct((P * 8, COUT), jnp.float32),
        interpret=interpret,
    )()


_QUARTER = NPT // 4
_TRASH = 16384
_SHALF = SLOTS // 2
_L = 16


def _sc_mesh():
    return plsc.VectorSubcoreMesh(core_axis_name="c", subcore_axis_name="s")


def _iota16():
    return lax.iota(jnp.int32, 16)


def _phase_S1_sc(pid_pad):
    zc = jnp.zeros((16200,), jnp.int32)
    on = jnp.ones((128,), jnp.int32)

    @functools.partial(
        pl.kernel, mesh=_sc_mesh(),
        out_type=jax.ShapeDtypeStruct((NPT,), jnp.int32),
        scratch_types=[
            pltpu.VMEM_SHARED((_QUARTER + _TRASH,), jnp.int32),
            pltpu.VMEM((2560,), jnp.int32),
            pltpu.VMEM((2560,), jnp.int32),
            pltpu.VMEM((16200,), jnp.int32),
            pltpu.VMEM((16200,), jnp.int32),
            pltpu.VMEM((128,), jnp.int32),
        ],
    )
    def k(pid_hbm, zc_hbm, on_hbm, occ_hbm, counts, pidv, idxv, zbuf, wbuf,
          ones):
        c = lax.axis_index("c")
        t = lax.axis_index("s")
        pltpu.sync_copy(zc_hbm, zbuf)
        pltpu.sync_copy(on_hbm, ones)
        pltpu.sync_copy(pid_hbm.at[pl.ds(t * 2560, 2560)], pidv)
        for half in range(2):
            base = half * 2 * _QUARTER + c * _QUARTER
            for j in range(4):
                pltpu.sync_copy(zbuf, counts.at[pl.ds(t * 64800 + j * 16200,
                                                      16200)])
            plsc.subcore_barrier()
            def mk(k_, _):
                vec = pidv[pl.ds(k_ * 16, 16)]
                local = vec - base
                inr = (local >= 0) & (local < _QUARTER)
                sp = _QUARTER + ((k_ * 16 + _iota16()) & (_TRASH - 1))
                idx = jnp.where(inr, local, sp)
                idxv[pl.ds(k_ * 16, 16)] = idx
                return 0
            lax.fori_loop(0, 160, mk, 0, unroll=8)
            for ck in range(20):
                pltpu.sync_copy(ones,
                                counts.at[idxv.at[pl.ds(ck * 128, 128)]],
                                add=True)
            plsc.subcore_barrier()
            for j in range(4):
                sl = pl.ds(t * 64800 + j * 16200, 16200)
                pltpu.sync_copy(counts.at[sl], wbuf)
                pltpu.sync_copy(
                    wbuf, occ_hbm.at[pl.ds(base + t * 64800 + j * 16200,
                                           16200)])
            if half == 0:
                plsc.subcore_barrier()

    return k(pid_pad, zc, on)


def _phase_S3_sc(ypref, pid_pad, zi):
    zc32 = jnp.zeros((16384,), jnp.float32)
    on32 = jnp.ones((128,), jnp.float32)

    @functools.partial(
        pl.kernel, mesh=_sc_mesh(),
        out_type=[jax.ShapeDtypeStruct((SLOTS,), jnp.float32),
                  jax.ShapeDtypeStruct((PPAD,), jnp.int32)],
        scratch_types=[
            pltpu.VMEM_SHARED((_SHALF + _TRASH,), jnp.float32),
            pltpu.VMEM((2560,), jnp.int32),
            pltpu.VMEM((2560,), jnp.int32),
            pltpu.VMEM((2560,), jnp.int32),
            pltpu.VMEM((2560,), jnp.int32),
            pltpu.VMEM((128,), jnp.float32),
            pltpu.VMEM((16384,), jnp.float32),
            pltpu.VMEM((128,), jnp.float32),
        ],
    )
    def k(yp_hbm, pid_hbm, zi_hbm, zc_hbm, on_hbm, cs_hbm, s_hbm, slots,
          pidv, ziv, sv, sidx, ypv, zbuf, ones):
        c = lax.axis_index("c")
        t = lax.axis_index("s")
        pltpu.sync_copy(zc_hbm, zbuf)
        pltpu.sync_copy(on_hbm, ones)
        pltpu.sync_copy(zbuf, slots.at[pl.ds(t * 16384, 16384)])
        plsc.subcore_barrier()
        pltpu.sync_copy(pid_hbm.at[pl.ds(t * 2560, 2560)], pidv)
        pltpu.sync_copy(zi_hbm.at[pl.ds(t * 2560, 2560)], ziv)
        base = t * 2560
        half_lo = c * _SHALF
        def clampg(l, _):
            vec = pidv[pl.ds(l * 16, 16)]
            pidv[pl.ds(l * 16, 16)] = jnp.minimum(vec, NPT - 1)
            return 0
        lax.fori_loop(0, 160, clampg, 0, unroll=8)
        for ck in range(20):
            pltpu.sync_copy(yp_hbm.at[pidv.at[pl.ds(ck * 128, 128)]], ypv)
            def mks(l, _):
                y = ypv[pl.ds(l * 16, 16)]
                z = ziv[pl.ds(ck * 128 + l * 16, 16)]
                s = y.astype(jnp.int32) * 8 + z
                gpos = base + ck * 128 + l * 16 + _iota16()
                s = jnp.where(gpos < P,
                              s, NREAL_SLOTS + 8 + (gpos & 8191))
                sv[pl.ds(ck * 128 + l * 16, 16)] = s
                local = s - half_lo
                inr = (local >= 0) & (local < _SHALF)
                sp = _SHALF + ((ck * 128 + l * 16 + _iota16())
                               & (_TRASH - 1))
                sidx[pl.ds(ck * 128 + l * 16, 16)] = jnp.where(inr, local, sp)
                return 0
            lax.fori_loop(0, 8, mks, 0, unroll=8)
            pltpu.sync_copy(ones, slots.at[sidx.at[pl.ds(ck * 128, 128)]],
                            add=True)

        @pl.when(c == 0)
        def _():
            pltpu.sync_copy(sv, s_hbm.at[pl.ds(t * 2560, 2560)])
        plsc.subcore_barrier()
        sl = pl.ds(t * 16384, 16384)
        pltpu.sync_copy(slots.at[sl], zbuf)
        pltpu.sync_copy(zbuf, cs_hbm.at[pl.ds(c * _SHALF + t * 16384,
                                              16384)])

    return k(ypref, pid_pad, zi, zc32, on32)


def _phase_S5a_sc(s, cslots, vp2, sf2):
    @functools.partial(
        pl.kernel, mesh=_sc_mesh(),
        out_type=[jax.ShapeDtypeStruct((PPAD, 128), jnp.float32),
                  jax.ShapeDtypeStruct((PPAD,), jnp.float32),
                  jax.ShapeDtypeStruct((PPAD,), jnp.int32),
                  jax.ShapeDtypeStruct((PPAD,), jnp.int32),
                  jax.ShapeDtypeStruct((PPAD,), jnp.int32)],
        scratch_types=[
            pltpu.VMEM((1280,), jnp.int32),
            pltpu.VMEM((1280,), jnp.int32),
            pltpu.VMEM((1280,), jnp.int32),
            pltpu.VMEM((1280,), jnp.int32),
            pltpu.VMEM((1280,), jnp.int32),
            pltpu.VMEM((1280,), jnp.int32),
            pltpu.VMEM((128, 128), jnp.float32),
            pltpu.VMEM((128,), jnp.float32),
            pltpu.VMEM((1280,), jnp.float32),
        ],
    )
    def k(s_hbm, cs_hbm, vp2_hbm, sf2_hbm, sfg_hbm, rcp_hbm, parp_hbm,
          pars_hbm, v2_hbm, sv, rowv, spv, vv, parpv, parsv, rows, val, rcp):
        c = lax.axis_index("c")
        t = lax.axis_index("s")
        w = t * 2 + c
        b0 = w * 1280
        pltpu.sync_copy(s_hbm.at[pl.ds(b0, 1280)], sv)

        def prep(l, _):
            vec = sv[pl.ds(l * 16, 16)]
            pr = jnp.clip((vec >> 3) - 1, 0, P - 1)
            rowv[pl.ds(l * 16, 16)] = pr >> 1
            parpv[pl.ds(l * 16, 16)] = pr & 1
            spv[pl.ds(l * 16, 16)] = vec >> 1
            parsv[pl.ds(l * 16, 16)] = vec & 1
            return 0
        lax.fori_loop(0, 80, prep, 0, unroll=8)
        for ck in range(10):
            pltpu.sync_copy(sf2_hbm.at[rowv.at[pl.ds(ck * 128, 128)]], rows)
            pltpu.sync_copy(rows, sfg_hbm.at[pl.ds(b0 + ck * 128, 128)])
            pltpu.sync_copy(cs_hbm.at[sv.at[pl.ds(ck * 128, 128)]], val)

            def mkrcp(l, _):
                d = val[pl.ds(l * 16, 16)]
                rcp[pl.ds(ck * 128 + l * 16, 16)] = 1.0 / jnp.maximum(d, 1.0)
                return 0
            lax.fori_loop(0, 8, mkrcp, 0, unroll=8)
            pltpu.sync_copy(vp2_hbm.at[spv.at[pl.ds(ck * 128, 128)]], val)

            def mkv(l, _):
                vv[pl.ds(ck * 128 + l * 16, 16)] = (
                    val[pl.ds(l * 16, 16)].astype(jnp.int32) - 1)
                return 0
            lax.fori_loop(0, 8, mkv, 0, unroll=8)
        pltpu.sync_copy(vv, v2_hbm.at[pl.ds(b0, 1280)])
        pltpu.sync_copy(rcp, rcp_hbm.at[pl.ds(b0, 1280)])
        pltpu.sync_copy(parpv, parp_hbm.at[pl.ds(b0, 1280)])
        pltpu.sync_copy(parsv, pars_hbm.at[pl.ds(b0, 1280)])

    sfg2, rcpf, parp, pars, v2 = k(s, cslots, vp2, sf2)
    return (sfg2, rcpf.reshape(PPAD, 1), parp.reshape(PPAD, 1),
            pars.reshape(PPAD, 1), v2)


def _phase_S5c_sc(fs2, v2):
    _VQ = VCAP2 // 4
    zc2 = jnp.zeros((72, 128), jnp.float32)

    @functools.partial(
        pl.kernel, mesh=_sc_mesh(),
        out_type=jax.ShapeDtypeStruct((VCAP2, 128), jnp.float32),
        scratch_types=[
            pltpu.VMEM_SHARED((_VQ + 64, 128), jnp.float32),
            pltpu.VMEM((2560,), jnp.int32),
            pltpu.VMEM((2560,), jnp.int32),
            pltpu.VMEM((72, 128), jnp.float32),
            pltpu.VMEM((72, 128), jnp.float32),
            pltpu.VMEM((128, 128), jnp.float32),
        ],
    )
    def k(fs_hbm, v_hbm, zc_hbm, vox_hbm, acc, vv, idxv, zb, wb, rows):
        c = lax.axis_index("c")
        t = lax.axis_index("s")
        pltpu.sync_copy(zc_hbm, zb)
        pltpu.sync_copy(v_hbm.at[pl.ds(t * 2560, 2560)], vv)
        for ps in range(2):
            qbase = (2 * ps + c) * _VQ
            for j in range(9):
                pltpu.sync_copy(zb, acc.at[pl.ds(t * 648 + j * 72, 72)])
            plsc.subcore_barrier()

            def mk(l, _):
                vec = vv[pl.ds(l * 16, 16)]
                local = vec - qbase
                inr = (local >= 0) & (local < _VQ)
                sp = _VQ + ((l * 16 + _iota16()) & 63)
                idxv[pl.ds(l * 16, 16)] = jnp.where(inr, local, sp)
                return 0
            lax.fori_loop(0, 160, mk, 0, unroll=8)
            for ck in range(20):
                pltpu.sync_copy(fs_hbm.at[pl.ds(t * 2560 + ck * 128, 128)],
                                rows)
                pltpu.sync_copy(rows, acc.at[idxv.at[pl.ds(ck * 128, 128)]],
                                add=True)
            plsc.subcore_barrier()
            for j in range(9):
                pltpu.sync_copy(acc.at[pl.ds(t * 648 + j * 72, 72)], wb)
                pltpu.sync_copy(wb, vox_hbm.at[pl.ds(qbase + t * 648
                                                     + j * 72, 72)])
            if ps == 0:
                plsc.subcore_barrier()

    return k(fs2, v2, zc2)


def _phase_S6_sc(cp_shifted, vp2_shifted, vox2):
    _NCH = (P * 8 // 2) // 128

    @functools.partial(
        pl.kernel, mesh=_sc_mesh(),
        out_type=jax.ShapeDtypeStruct((P * 8 // 2, 128), jnp.float32),
        scratch_types=[
            pltpu.VMEM((2, 128), jnp.float32),
            pltpu.VMEM((2, 128), jnp.float32),
            pltpu.VMEM((128,), jnp.int32),
            pltpu.VMEM((2, 128, 128), jnp.float32),
            pltpu.SemaphoreType.DMA,
            pltpu.SemaphoreType.DMA,
            pltpu.SemaphoreType.DMA,
        ],
    )
    def k(cp_hbm, vp_hbm, vox_hbm, out_hbm, cpv, vpv, idxv, rows,
          psem, gsem, wsem):
        c = lax.axis_index("c")
        t = lax.axis_index("s")
        w = t * 2 + c

        def cof(kk):
            return lax.rem(w + kk * 32, _NCH) * 128

        def load(kk, pg):
            a = pltpu.async_copy(cp_hbm.at[pl.ds(cof(kk), 128)],
                                 cpv.at[pg], psem)
            b = pltpu.async_copy(vp_hbm.at[pl.ds(cof(kk), 128)],
                                 vpv.at[pg], psem)
            return a, b

        hd = load(0, 0)
        prev_w = None
        for kk in range(40):
            pg = kk & 1
            r0 = cof(kk)
            nxt = load(kk + 1, 1 - pg) if kk + 1 < 40 else None
            hd[0].wait()
            hd[1].wait()

            def mk(l, _2):
                occ = cpv[pg, pl.ds(l * 16, 16)] > 0.0
                vr = vpv[pg, pl.ds(l * 16, 16)].astype(jnp.int32) - 1
                vr = jnp.clip(vr, 0, VCAP2 - 1)
                pos = r0 + l * 16 + _iota16()
                sp = 40000 + ((pos * 7 + l) & 1023)
                idxv[pl.ds(l * 16, 16)] = jnp.where(occ, vr, sp)
                return 0
            lax.fori_loop(0, 8, mk, 0, unroll=8)
            if prev_w is not None:
                prev_w.wait()
            pltpu.async_copy(vox_hbm.at[idxv], rows.at[pg], gsem).wait()
            prev_w = pltpu.async_copy(rows.at[pg],
                                      out_hbm.at[pl.ds(r0, 128)], wsem)
            hd = nxt
        prev_w.wait()

    return k(cp_shifted, vp2_shifted, vox2)


def _phase_T0_jnp(points_pad, W, gamma, beta):
    x = points_pad[:, 1:9]
    h = x @ W.T
    rowmask = (jnp.arange(PPAD) < P)
    hm = jnp.where(rowmask[:, None], h, 0.0)
    s1 = hm.sum(axis=0)
    s2 = (hm * hm).sum(axis=0)
    mu = s1 / P
    var = s2 / P - mu * mu
    g = gamma / jnp.sqrt(var + 1e-3)
    b2 = beta - g * mu
    zi = jnp.floor((points_pad[:, 6] + 2.0) / 0.5).astype(jnp.int32)
    zi = jnp.where(rowmask, zi, 0)
    coef = jnp.zeros((8, COUT), jnp.float32).at[0].set(g).at[1].set(b2)
    return h, coef, zi


def _phase_S1_jnp(pid_pad):
    occ = jnp.zeros((NPT,), jnp.int32).at[pid_pad].add(1)
    return occ


def _phase_T2_jnp(occ):
    return jnp.cumsum((occ != 0).astype(jnp.float32))


def _phase_S3_jnp(ypref, pid_pad, zi):
    yp = ypref[pid_pad]
    s = (yp.astype(jnp.int32) * 8 + zi)
    gpos = jnp.arange(PPAD)
    s = jnp.where(gpos < P, s, NREAL_SLOTS + 8 + (gpos - P))
    cslots = jnp.zeros((SLOTS,), jnp.float32).at[s].add(1.0)
    return cslots, s


def _phase_T4_jnp(cslots):
    occb = (cslots != 0).astype(jnp.float32)
    vrankp = jnp.cumsum(occb)
    occ8 = occb.reshape(SLOTS // 8, 8).sum(axis=1)
    return vrankp, occ8


def _phase_S5a_jnp(s, cslots, vrankp, sparse_feat):
    prank = (s >> 3) - 1
    prank_c = jnp.clip(prank, 0, P - 1)
    sfg = sparse_feat[prank_c]
    den = cslots[s]
    rcp = 1.0 / jnp.maximum(den, 1.0)
    den64 = jnp.broadcast_to(rcp[:, None], (PPAD, COUT))
    v = vrankp[s].astype(jnp.int32) - 1
    return sfg, den64, v


def _phase_T0B_jnp(h, coef, sfg, den64):
    g = coef[0]
    b2 = coef[1]
    bnr = jnp.maximum(g[None, :] * h + b2[None, :], 0.0)
    fs = (bnr + sfg) * den64
    rowmask = (jnp.arange(PPAD) < P)
    return jnp.where(rowmask[:, None], fs, 0.0)


def _phase_S5c_jnp(fs, v):
    vc = jnp.clip(v, 0, VCAP - 1)
    return jnp.zeros((VCAP, COUT), jnp.float32).at[vc].add(fs)


def _phase_S6_jnp(vox, s, v, shift, out0):
    gpos = jnp.arange(PPAD)
    lidx = jnp.minimum(gpos, P - 1)
    s_c = s[lidx]
    v_c = v[lidx]
    f = s_c - 8 * (1 - shift)
    rows = vox[jnp.clip(v_c, 0, VCAP - 1)]
    out = out0.at[f].set(rows)
    return out


def kernel(points_with_f_center, pillar_merge_coords, sparse_feat,
           points_indices_inv, W, gamma, beta):
    pts_pad = jnp.concatenate(
        [points_with_f_center,
         jnp.zeros((PPAD - P, 9), jnp.float32)], axis=0)
    pid_fill = jnp.full((PPAD - P,), 1 << 29, jnp.int32)
    pid_pad = jnp.concatenate(
        [pillar_merge_coords.astype(jnp.int32), pid_fill], axis=0)

    h, zi3d, coef = _phase_T0_pallas(pts_pad, W, gamma, beta)
    zi = zi3d.reshape(PPAD)
    occ = _phase_S1_sc(pid_pad)
    y2d, _ = _prefix_pallas(occ.reshape(NPT // 256, 256), 600)
    ypref = y2d.reshape(NPT)
    cslots, s = _phase_S3_sc(ypref, pid_pad, zi)
    v2d, o82d = _prefix_pallas(cslots.reshape(SLOTS // 256, 256), 1024)
    vrankp = v2d.reshape(SLOTS)
    occ8 = o82d.reshape(SLOTS // 8)

    nv = vrankp[NREAL_SLOTS - 1].astype(jnp.int32)
    shift = jnp.where(nv < P, 1, 0).astype(jnp.int32)

    cpair = cslots.reshape(SLOTS // 2, 2).sum(axis=1)
    vp2_2d, _ = _prefix_pallas(cpair.reshape(SLOTS // 512, 256), 1024)
    vp2 = vp2_2d.reshape(SLOTS // 2)
    sf2 = sparse_feat.reshape(P // 2, 128)

    sfg2, rcp2d, parp2d, pars2d, v2 = _phase_S5a_sc(s, cslots, vp2, sf2)
    fs2 = _phase_T0B_pallas(h, coef, sfg2, rcp2d, parp2d, pars2d)
    vox2 = _phase_S5c_sc(fs2, v2)
    d2 = 4 * (1 - shift)
    cp_sh = lax.dynamic_slice(cpair, (d2,), (P * 4,))
    vp2_sh = lax.dynamic_slice(vp2, (d2,), (P * 4,))
    out = _phase_S6_sc(cp_sh, vp2_sh, vox2)

    src = out.reshape(P, 8, COUT)
    m1 = occ8[:P] >= 2
    m1 = m1.at[0].set((P - nv) >= 2)
    m0 = occ8[1:P + 1] >= 2
    occupied_mask = jnp.where(shift == 1, m1, m0)
    return src, occupied_mask
